# Initial kernel scaffold; baseline (speedup 1.0000x reference)
import jax
import jax.numpy as jnp
from jax.experimental import pallas as pl


def kernel(mixed_vcf, ref_panel, weights):
    raise NotImplementedError("write your pallas kernel here")


# speedup vs baseline: 5.0594x; 5.0594x over previous
import functools

import jax
import jax.numpy as jnp
from jax import lax
from jax.experimental import pallas as pl
from jax.experimental.pallas import tpu as pltpu
from jax.experimental.pallas import tpu_sc as plsc

_B, _A, _R, _L = 4, 2, 64, 65536
_BA = _B * _A
_NC, _NS, _LANES = 2, 16, 16
_NW = _NC * _NS
_WPR = _NW // _BA
_LW = _L // _WPR
_C = 512
_NCHUNK = _LW // _C


def _sc_call(mixed, ref, wpad):
    mesh = plsc.VectorSubcoreMesh(core_axis_name="c", subcore_axis_name="s")

    @functools.partial(
        pl.kernel,
        mesh=mesh,
        out_type=[
            jax.ShapeDtypeStruct((_BA, _L), jnp.float32),
            jax.ShapeDtypeStruct((_BA, _L), jnp.int32),
        ],
        scratch_types=[
            pltpu.VMEM((_R, _C), jnp.float32),
            pltpu.VMEM((_C,), jnp.float32),
            pltpu.VMEM((_C,), jnp.float32),
            pltpu.VMEM((_C,), jnp.int32),
            pltpu.VMEM((_LANES,), jnp.float32),
        ],
    )
    def body(mixed_hbm, ref_hbm, w_hbm, pooled_hbm, idx_hbm,
             ref_v, m_v, p_v, i_v, w_v):
        wid = lax.axis_index("s") * _NC + lax.axis_index("c")
        ba = wid // _WPR
        b = ba // _A
        l_base = (wid % _WPR) * _LW

        pltpu.sync_copy(w_hbm, w_v)
        wvec = w_v[...]
        w0 = wvec[0]
        w1 = wvec[1]

        def chunk_body(ci, _):
            l0 = l_base + ci * _C
            pltpu.sync_copy(ref_hbm.at[ba, :, pl.ds(l0, _C)], ref_v)
            pltpu.sync_copy(mixed_hbm.at[b, pl.ds(l0, _C)], m_v)

            def j_body(j, _):
                m = m_v[pl.ds(j * _LANES, _LANES)]

                def r_body(r, carry):
                    mx1, mx2, ix = carry
                    v = ref_v[r, pl.ds(j * _LANES, _LANES)] * m
                    gt = v > mx1
                    mx2n = jnp.where(gt, mx1, jnp.maximum(mx2, v))
                    ixn = jnp.where(gt, jnp.full((_LANES,), 0, jnp.int32) + r, ix)
                    mx1n = jnp.where(gt, v, mx1)
                    return mx1n, mx2n, ixn

                neg = jnp.full((_LANES,), -jnp.inf, jnp.float32)
                mx1, mx2, ix = lax.fori_loop(
                    0, _R, r_body,
                    (neg, neg, jnp.zeros((_LANES,), jnp.int32)))
                p_v[pl.ds(j * _LANES, _LANES)] = mx1 * w0 + mx2 * w1
                i_v[pl.ds(j * _LANES, _LANES)] = ix
                return 0

            lax.fori_loop(0, _C // _LANES, j_body, 0)
            pltpu.sync_copy(p_v, pooled_hbm.at[ba, pl.ds(l0, _C)])
            pltpu.sync_copy(i_v, idx_hbm.at[ba, pl.ds(l0, _C)])
            return 0

        lax.fori_loop(0, _NCHUNK, chunk_body, 0)

    return body(mixed, ref, wpad)


def kernel(mixed_vcf, ref_panel, weights):
    ref = ref_panel.reshape(_BA, _R, _L)
    k = weights.shape[0]
    wpad = jnp.pad(weights.reshape(-1), (0, _LANES - k))
    pooled, idx = _sc_call(mixed_vcf, ref, wpad)
    return pooled.reshape(_B, _A, _L), idx.reshape(_B, _A, _L)


# speedup vs baseline: 17.5835x; 3.4754x over previous
import functools

import jax
import jax.numpy as jnp
from jax import lax
from jax.experimental import pallas as pl
from jax.experimental.pallas import tpu as pltpu
from jax.experimental.pallas import tpu_sc as plsc

_B, _A, _R, _L = 4, 2, 64, 65536
_BA = _B * _A
_NC, _NS, _LANES = 2, 16, 16
_NW = _NC * _NS
_WPR = _NW // _BA
_LW = _L // _WPR
_C = 512
_NCHUNK = _LW // _C


def _sc_call(mixed, ref, wpad):
    mesh = plsc.VectorSubcoreMesh(core_axis_name="c", subcore_axis_name="s")

    @functools.partial(
        pl.kernel,
        mesh=mesh,
        out_type=[
            jax.ShapeDtypeStruct((_BA, _L), jnp.float32),
            jax.ShapeDtypeStruct((_BA, _L), jnp.int32),
        ],
        scratch_types=[
            pltpu.VMEM((2, _R, _C), jnp.float32),
            pltpu.VMEM((_LW,), jnp.float32),
            pltpu.VMEM((_LW,), jnp.float32),
            pltpu.VMEM((_LW,), jnp.int32),
            pltpu.VMEM((_LANES,), jnp.float32),
            pltpu.SemaphoreType.DMA,
            pltpu.SemaphoreType.DMA,
        ],
    )
    def body(mixed_hbm, ref_hbm, w_hbm, pooled_hbm, idx_hbm,
             ref_v, m_v, p_v, i_v, w_v, sem0, sem1):
        wid = lax.axis_index("s") * _NC + lax.axis_index("c")
        ba = wid // _WPR
        b = ba // _A
        l_base = (wid % _WPR) * _LW
        sems = (sem0, sem1)

        def start_in(ci, par):
            l0 = l_base + ci * _C
            pltpu.async_copy(
                ref_hbm.at[ba, :, pl.ds(l0, _C)], ref_v.at[par], sems[par])

        def wait_in(par):
            pltpu.make_async_copy(
                ref_hbm.at[ba, :, pl.ds(l_base, _C)], ref_v.at[par],
                sems[par]).wait()

        pltpu.sync_copy(w_hbm, w_v)
        pltpu.sync_copy(mixed_hbm.at[b, pl.ds(l_base, _LW)], m_v)
        wvec = w_v[...]
        w0 = wvec[0]
        w1 = wvec[1]

        start_in(0, 0)
        start_in(1, 1)

        def compute(ci, par):
            off0 = ci * _C

            def j_body(j, _):
                off = off0 + j * _LANES
                m = m_v[pl.ds(off, _LANES)]

                def r_body(r, carry):
                    mx1, mx2, ix = carry
                    v = ref_v[par, r, pl.ds(j * _LANES, _LANES)] * m
                    gt = v > mx1
                    mx2n = jnp.where(gt, mx1, jnp.maximum(mx2, v))
                    ixn = jnp.where(gt, jnp.full((_LANES,), 0, jnp.int32) + r, ix)
                    mx1n = jnp.where(gt, v, mx1)
                    return mx1n, mx2n, ixn

                neg = jnp.full((_LANES,), -jnp.inf, jnp.float32)
                mx1, mx2, ix = lax.fori_loop(
                    0, _R, r_body,
                    (neg, neg, jnp.zeros((_LANES,), jnp.int32)),
                    unroll=8)
                p_v[pl.ds(off, _LANES)] = mx1 * w0 + mx2 * w1
                i_v[pl.ds(off, _LANES)] = ix
                return 0

            lax.fori_loop(0, _C // _LANES, j_body, 0)

        def chunk_pair(cp, _):
            ci0 = cp * 2
            wait_in(0)
            compute(ci0, 0)

            @pl.when(ci0 + 2 < _NCHUNK)
            def _():
                start_in(ci0 + 2, 0)

            wait_in(1)
            compute(ci0 + 1, 1)

            @pl.when(ci0 + 3 < _NCHUNK)
            def _():
                start_in(ci0 + 3, 1)

            return 0

        lax.fori_loop(0, _NCHUNK // 2, chunk_pair, 0)

        pltpu.sync_copy(p_v, pooled_hbm.at[ba, pl.ds(l_base, _LW)])
        pltpu.sync_copy(i_v, idx_hbm.at[ba, pl.ds(l_base, _LW)])

    return body(mixed, ref, wpad)


def kernel(mixed_vcf, ref_panel, weights):
    ref = ref_panel.reshape(_BA, _R, _L)
    k = weights.shape[0]
    wpad = jnp.pad(weights.reshape(-1), (0, _LANES - k))
    pooled, idx = _sc_call(mixed_vcf, ref, wpad)
    return pooled.reshape(_B, _A, _L), idx.reshape(_B, _A, _L)
